# baseline (speedup 1.0000x reference)
import jax
import jax.numpy as jnp
from jax import lax
from jax.experimental import pallas as pl
from jax.experimental.pallas import tpu as pltpu
from jax.experimental.pallas import tpu_sc as plsc

NC = 2
NS = 16
NW = NC * NS

ROWS_W = 128
HALF = 64
UNITS = 50 * (ROWS_W // HALF)


def _body(table_hbm, idx_hbm, out_hbm, idx_v, st0, st1, g0, g1, w0, w1):
    wid = lax.axis_index("s") * NC + lax.axis_index("c")
    col0 = wid * ROWS_W
    stage = (st0, st1)
    gsem = (g0, g1)
    wsem = (w0, w1)

    pltpu.sync_copy(idx_hbm.at[wid], idx_v)

    def dst_of(u):
        j = u // 2
        h = u % 2
        return out_hbm.at[j, pl.ds(col0 + h * HALF, HALF)]

    pltpu.async_copy(table_hbm.at[idx_v.at[0]], st0, g0)
    pltpu.async_copy(table_hbm.at[idx_v.at[1]], st1, g1)

    def step(g, carry):
        for b in range(2):
            u = g * 2 + b
            pltpu.make_async_copy(table_hbm.at[idx_v.at[u]], stage[b], gsem[b]).wait()
            dst = dst_of(u)
            pltpu.async_copy(stage[b], dst, wsem[b])

            @pl.when(u + 2 < UNITS)
            def _():
                pltpu.make_async_copy(stage[b], dst, wsem[b]).wait()
                pltpu.async_copy(table_hbm.at[idx_v.at[u + 2]], stage[b], gsem[b])

        return carry

    lax.fori_loop(0, UNITS // 2, step, 0)

    pltpu.make_async_copy(st0, dst_of(UNITS - 2), w0).wait()
    pltpu.make_async_copy(st1, dst_of(UNITS - 1), w1).wait()


def kernel(ids, kernel):
    table = kernel
    n_rows, d = table.shape
    nb_rows, seq = ids.shape
    assert nb_rows == NW * ROWS_W

    idx = (
        ids.astype(jnp.int32)
        .T.reshape(seq, NW, ROWS_W // HALF, HALF)
        .transpose(1, 0, 2, 3)
        .reshape(NW, UNITS, HALF)
    )
    idx = idx + ((jnp.arange(NW, dtype=jnp.int32) // 8) * n_rows)[:, None, None]
    table_rep = jnp.tile(table, (NW // 8, 1))

    mesh = plsc.VectorSubcoreMesh(
        core_axis_name="c", subcore_axis_name="s", num_cores=NC, num_subcores=NS
    )
    run = pl.kernel(
        _body,
        out_type=jax.ShapeDtypeStruct((seq, nb_rows, d), table.dtype),
        mesh=mesh,
        scratch_types=[
            pltpu.VMEM((UNITS, HALF), jnp.int32),
            pltpu.VMEM((HALF, d), jnp.float32),
            pltpu.VMEM((HALF, d), jnp.float32),
            pltpu.SemaphoreType.DMA,
            pltpu.SemaphoreType.DMA,
            pltpu.SemaphoreType.DMA,
            pltpu.SemaphoreType.DMA,
        ],
    )
    out3 = run(table_rep, idx)
    return out3.transpose(1, 0, 2)


# speedup vs baseline: 1.0424x; 1.0424x over previous
import jax
import jax.numpy as jnp
from jax import lax
from jax.experimental import pallas as pl
from jax.experimental.pallas import tpu as pltpu
from jax.experimental.pallas import tpu_sc as plsc

NC = 2
NS = 16
NW = NC * NS

ROWS_W = 128
HALF = 64
UNITS = 50 * (ROWS_W // HALF)
NBUF = 3
UNITS_EFF = 102


def _body(table_hbm, idx_hbm, out_hbm, idx_v, st0, st1, st2, g0, g1, g2, w0, w1, w2):
    wid = lax.axis_index("s") * NC + lax.axis_index("c")
    col0 = wid * ROWS_W
    stage = (st0, st1, st2)
    gsem = (g0, g1, g2)
    wsem = (w0, w1, w2)

    pltpu.sync_copy(idx_hbm.at[wid], idx_v)

    def dst_of(u):
        ur = lax.rem(u, UNITS)
        j = ur // 2
        h = lax.rem(ur, 2)
        return out_hbm.at[j, pl.ds(col0 + h * HALF, HALF)]

    for b in range(NBUF):
        pltpu.async_copy(table_hbm.at[idx_v.at[b]], stage[b], gsem[b])

    def step(g, carry):
        for b in range(NBUF):
            u = g * NBUF + b
            pltpu.make_async_copy(table_hbm.at[idx_v.at[u]], stage[b], gsem[b]).wait()
            dst = dst_of(u)
            pltpu.async_copy(stage[b], dst, wsem[b])

            @pl.when(u + NBUF < UNITS_EFF)
            def _():
                pltpu.make_async_copy(stage[b], dst, wsem[b]).wait()
                pltpu.async_copy(table_hbm.at[idx_v.at[u + NBUF]], stage[b], gsem[b])

        return carry

    lax.fori_loop(0, UNITS_EFF // NBUF, step, 0)

    for b in range(NBUF):
        u = UNITS_EFF - NBUF + b
        pltpu.make_async_copy(stage[b], dst_of(u), wsem[b]).wait()


def kernel(ids, kernel):
    table = kernel
    n_rows, d = table.shape
    nb_rows, seq = ids.shape
    assert nb_rows == NW * ROWS_W

    idx = (
        ids.astype(jnp.int32)
        .T.reshape(seq, NW, ROWS_W // HALF, HALF)
        .transpose(1, 0, 2, 3)
        .reshape(NW, UNITS, HALF)
    )
    idx = jnp.concatenate([idx, idx[:, : UNITS_EFF - UNITS, :]], axis=1)
    idx = idx + ((jnp.arange(NW, dtype=jnp.int32) // 4) * n_rows)[:, None, None]
    table_rep = jnp.tile(table, (NW // 4, 1))

    mesh = plsc.VectorSubcoreMesh(
        core_axis_name="c", subcore_axis_name="s", num_cores=NC, num_subcores=NS
    )
    run = pl.kernel(
        _body,
        out_type=jax.ShapeDtypeStruct((seq, nb_rows, d), table.dtype),
        mesh=mesh,
        scratch_types=[
            pltpu.VMEM((UNITS_EFF, HALF), jnp.int32),
            pltpu.VMEM((HALF, d), jnp.float32),
            pltpu.VMEM((HALF, d), jnp.float32),
            pltpu.VMEM((HALF, d), jnp.float32),
            pltpu.SemaphoreType.DMA,
            pltpu.SemaphoreType.DMA,
            pltpu.SemaphoreType.DMA,
            pltpu.SemaphoreType.DMA,
            pltpu.SemaphoreType.DMA,
            pltpu.SemaphoreType.DMA,
        ],
    )
    out3 = run(table_rep, idx)
    return out3.transpose(1, 0, 2)


# speedup vs baseline: 1.0697x; 1.0261x over previous
import jax
import jax.numpy as jnp
from jax import lax
from jax.experimental import pallas as pl
from jax.experimental.pallas import tpu as pltpu
from jax.experimental.pallas import tpu_sc as plsc

NC = 2
NS = 16
NW = NC * NS

ROWS_W = 128
HALF = 64
UNITS = 50 * (ROWS_W // HALF)


def _body(table_hbm, idx_hbm, out_hbm, idx_v, st0, st1, g0, g1, w0, w1):
    wid = lax.axis_index("s") * NC + lax.axis_index("c")
    col0 = wid * ROWS_W
    stage = (st0, st1)
    gsem = (g0, g1)
    wsem = (w0, w1)

    pltpu.sync_copy(idx_hbm.at[wid], idx_v)

    def dst_of(u):
        j = u // 2
        h = u % 2
        return out_hbm.at[j, pl.ds(col0 + h * HALF, HALF)]

    pltpu.async_copy(table_hbm.at[idx_v.at[0]], st0, g0)
    pltpu.async_copy(table_hbm.at[idx_v.at[1]], st1, g1)

    def step(g, carry):
        for b in range(2):
            u = g * 2 + b
            pltpu.make_async_copy(table_hbm.at[idx_v.at[u]], stage[b], gsem[b]).wait()
            dst = dst_of(u)
            pltpu.async_copy(stage[b], dst, wsem[b])

            @pl.when(u + 2 < UNITS)
            def _():
                pltpu.make_async_copy(stage[b], dst, wsem[b]).wait()
                pltpu.async_copy(table_hbm.at[idx_v.at[u + 2]], stage[b], gsem[b])

        return carry

    lax.fori_loop(0, UNITS // 2, step, 0)

    pltpu.make_async_copy(st0, dst_of(UNITS - 2), w0).wait()
    pltpu.make_async_copy(st1, dst_of(UNITS - 1), w1).wait()


def kernel(ids, kernel):
    table = kernel
    n_rows, d = table.shape
    nb_rows, seq = ids.shape
    assert nb_rows == NW * ROWS_W

    idx = (
        ids.astype(jnp.int32)
        .T.reshape(seq, NW, ROWS_W // HALF, HALF)
        .transpose(1, 0, 2, 3)
        .reshape(NW, UNITS, HALF)
    )
    idx = idx + ((jnp.arange(NW, dtype=jnp.int32) // 4) * n_rows)[:, None, None]
    table_rep = jnp.tile(table, (NW // 4, 1))

    mesh = plsc.VectorSubcoreMesh(
        core_axis_name="c", subcore_axis_name="s", num_cores=NC, num_subcores=NS
    )
    run = pl.kernel(
        _body,
        out_type=jax.ShapeDtypeStruct((seq, nb_rows, d), table.dtype),
        mesh=mesh,
        scratch_types=[
            pltpu.VMEM((UNITS, HALF), jnp.int32),
            pltpu.VMEM((HALF, d), jnp.float32),
            pltpu.VMEM((HALF, d), jnp.float32),
            pltpu.SemaphoreType.DMA,
            pltpu.SemaphoreType.DMA,
            pltpu.SemaphoreType.DMA,
            pltpu.SemaphoreType.DMA,
        ],
    )
    out3 = run(table_rep, idx)
    return out3.transpose(1, 0, 2)
